# baseline (speedup 1.0000x reference)
import functools

import jax
import jax.numpy as jnp
from jax import lax
from jax.experimental import pallas as pl
from jax.experimental.pallas import tpu as pltpu
from jax.experimental.pallas import tpu_sc as plsc

VOCAB = 100000
D = 64
BATCH = 1024
CTX = 10
LANES = 16

_INFO = plsc.get_sparse_core_info()
_NC, _NS = _INFO.num_cores, _INFO.num_subcores
_NW = _NC * _NS
_B_PER_W = BATCH // _NW
_IDS_PER_W = _B_PER_W * CTX
_GCHUNK = 80
_NGCHUNK = _IDS_PER_W // _GCHUNK


def _gather_mean_body(ids_hbm, table_hbm, out_hbm, idx_v, rows_v, mean_v, sem):
    wid = lax.axis_index("s") * _NC + lax.axis_index("c")
    base = wid * _IDS_PER_W
    pltpu.sync_copy(ids_hbm.at[pl.ds(base, _IDS_PER_W)], idx_v)
    copies = [
        pltpu.async_copy(
            table_hbm.at[idx_v.at[pl.ds(k * _GCHUNK, _GCHUNK)]],
            rows_v.at[pl.ds(k * _GCHUNK, _GCHUNK)],
            sem,
        )
        for k in range(_NGCHUNK)
    ]
    for c in copies:
        c.wait()

    def body(b, carry):
        row0 = b * CTX
        for c in range(D // LANES):
            sl = pl.ds(c * LANES, LANES)
            acc = rows_v[row0, sl]
            for j in range(1, CTX):
                acc = acc + rows_v[row0 + j, sl]
            mean_v[b, sl] = acc * jnp.float32(1.0 / CTX)
        return carry

    lax.fori_loop(0, _B_PER_W, body, 0)
    pltpu.sync_copy(mean_v, out_hbm.at[pl.ds(wid * _B_PER_W, _B_PER_W)])


_gather_mean = functools.partial(
    pl.kernel,
    out_type=jax.ShapeDtypeStruct((BATCH, D), jnp.float32),
    mesh=plsc.VectorSubcoreMesh(core_axis_name="c", subcore_axis_name="s"),
    scratch_types=[
        pltpu.VMEM((_IDS_PER_W,), jnp.int32),
        pltpu.VMEM((_IDS_PER_W, D), jnp.float32),
        pltpu.VMEM((_B_PER_W, D), jnp.float32),
        pltpu.SemaphoreType.DMA,
    ],
    compiler_params=pltpu.CompilerParams(use_tc_tiling_on_sc=False),
)(_gather_mean_body)


_VB = 2048


def _proj_body(x_ref, w_ref, out_ref):
    out_ref[...] = lax.dot_general(
        x_ref[...],
        w_ref[...].astype(jnp.bfloat16),
        (((1,), (1,)), ((), ())),
        preferred_element_type=jnp.float32,
    )


def _project(x, w):
    grid = (pl.cdiv(VOCAB, _VB),)
    return pl.pallas_call(
        _proj_body,
        grid=grid,
        in_specs=[
            pl.BlockSpec((BATCH, D), lambda i: (0, 0)),
            pl.BlockSpec((_VB, D), lambda i: (i, 0)),
        ],
        out_specs=pl.BlockSpec((BATCH, _VB), lambda i: (0, i)),
        out_shape=jax.ShapeDtypeStruct((BATCH, VOCAB), jnp.float32),
        compiler_params=pltpu.CompilerParams(
            dimension_semantics=("parallel",),
        ),
    )(x, w)


def kernel(context_ids, emb_table, out_weight):
    ids_flat = context_ids.reshape(BATCH * CTX).astype(jnp.int32)
    means = _gather_mean(ids_flat, emb_table)
    return _project(means.astype(jnp.bfloat16), out_weight)


# speedup vs baseline: 1.0619x; 1.0619x over previous
import functools

import jax
import jax.numpy as jnp
from jax import lax
from jax.experimental import pallas as pl
from jax.experimental.pallas import tpu as pltpu
from jax.experimental.pallas import tpu_sc as plsc

VOCAB = 100000
D = 64
BATCH = 1024
CTX = 10
LANES = 16

_INFO = plsc.get_sparse_core_info()
_NC, _NS = _INFO.num_cores, _INFO.num_subcores
_NW = _NC * _NS
_B_PER_W = BATCH // _NW
_IDS_PER_W = _B_PER_W * CTX
_GCHUNK = 80
_NGCHUNK = _IDS_PER_W // _GCHUNK


def _gather_mean_body(ids_hbm, table_hbm, out_hbm, idx_v, rows_v, mean_v, sem):
    wid = lax.axis_index("s") * _NC + lax.axis_index("c")
    base = wid * _IDS_PER_W
    pltpu.sync_copy(ids_hbm.at[pl.ds(base, _IDS_PER_W)], idx_v)
    copies = [
        pltpu.async_copy(
            table_hbm.at[idx_v.at[pl.ds(k * _GCHUNK, _GCHUNK)]],
            rows_v.at[pl.ds(k * _GCHUNK, _GCHUNK)],
            sem,
        )
        for k in range(_NGCHUNK)
    ]
    for c in copies:
        c.wait()

    def body(b, carry):
        row0 = b * CTX
        for c in range(D // LANES):
            sl = pl.ds(c * LANES, LANES)
            acc = rows_v[row0, sl]
            for j in range(1, CTX):
                acc = acc + rows_v[row0 + j, sl]
            mean_v[b, sl] = acc * jnp.float32(1.0 / CTX)
        return carry

    lax.fori_loop(0, _B_PER_W, body, 0)
    pltpu.sync_copy(mean_v, out_hbm.at[pl.ds(wid * _B_PER_W, _B_PER_W)])


_gather_mean = functools.partial(
    pl.kernel,
    out_type=jax.ShapeDtypeStruct((BATCH, D), jnp.float32),
    mesh=plsc.VectorSubcoreMesh(core_axis_name="c", subcore_axis_name="s"),
    scratch_types=[
        pltpu.VMEM((_IDS_PER_W,), jnp.int32),
        pltpu.VMEM((_IDS_PER_W, D), jnp.float32),
        pltpu.VMEM((_B_PER_W, D), jnp.float32),
        pltpu.SemaphoreType.DMA,
    ],
    compiler_params=pltpu.CompilerParams(use_tc_tiling_on_sc=False),
)(_gather_mean_body)


_MB = 32
_NSTEP = BATCH // _MB
_NBUF = 3


def _proj_body(x_ref, w_ref, out_hbm, buf, sem):
    i = pl.program_id(0)
    slot = lax.rem(i, _NBUF)

    @pl.when(i >= _NBUF)
    def _():
        pltpu.make_async_copy(
            buf.at[slot],
            out_hbm.at[pl.ds((i - _NBUF) * _MB, _MB)],
            sem.at[slot],
        ).wait()

    buf[slot] = lax.dot_general(
        x_ref[...],
        w_ref[...],
        (((1,), (0,)), ((), ())),
        preferred_element_type=jnp.float32,
    )

    pltpu.make_async_copy(
        buf.at[slot],
        out_hbm.at[pl.ds(i * _MB, _MB)],
        sem.at[slot],
    ).start()

    @pl.when(i == _NSTEP - 1)
    def _():
        for k in range(1, _NBUF):
            s = lax.rem(i - k + _NBUF, _NBUF)
            pltpu.make_async_copy(
                buf.at[s],
                out_hbm.at[pl.ds((i - k) * _MB, _MB)],
                sem.at[s],
            ).wait()
        pltpu.make_async_copy(
            buf.at[slot],
            out_hbm.at[pl.ds(i * _MB, _MB)],
            sem.at[slot],
        ).wait()


def _project(x, w):
    return pl.pallas_call(
        _proj_body,
        grid=(_NSTEP,),
        in_specs=[
            pl.BlockSpec((_MB, D), lambda i: (i, 0)),
            pl.BlockSpec((D, VOCAB), lambda i: (0, 0)),
        ],
        out_specs=pl.BlockSpec(memory_space=pl.ANY),
        out_shape=jax.ShapeDtypeStruct((BATCH, VOCAB), jnp.float32),
        scratch_shapes=[
            pltpu.VMEM((_NBUF, _MB, VOCAB), jnp.float32),
            pltpu.SemaphoreType.DMA((_NBUF,)),
        ],
        compiler_params=pltpu.CompilerParams(
            dimension_semantics=("arbitrary",),
            vmem_limit_bytes=100 * 1024 * 1024,
        ),
    )(x, w)


def kernel(context_ids, emb_table, out_weight):
    ids_flat = context_ids.reshape(BATCH * CTX).astype(jnp.int32)
    means = _gather_mean(ids_flat, emb_table)
    x = means.astype(jnp.bfloat16)
    return _project(x, out_weight.T.astype(jnp.bfloat16))


# speedup vs baseline: 2.7763x; 2.6144x over previous
import functools

import jax
import jax.numpy as jnp
from jax import lax
from jax.experimental import pallas as pl
from jax.experimental.pallas import tpu as pltpu
from jax.experimental.pallas import tpu_sc as plsc

VOCAB = 100000
D = 64
BATCH = 1024
CTX = 10
LANES = 16

_INFO = plsc.get_sparse_core_info()
_NC, _NS = _INFO.num_cores, _INFO.num_subcores
_NW = _NC * _NS
_B_PER_W = BATCH // _NW
_IDS_PER_W = _B_PER_W * CTX
_GCHUNK = 80
_NGCHUNK = _IDS_PER_W // _GCHUNK


def _gather_mean_body(ids_hbm, table_hbm, out_hbm, idx_v, rows_v, mean_v, sem):
    wid = lax.axis_index("s") * _NC + lax.axis_index("c")
    base = wid * _IDS_PER_W
    pltpu.sync_copy(ids_hbm.at[pl.ds(base, _IDS_PER_W)], idx_v)
    copies = [
        pltpu.async_copy(
            table_hbm.at[idx_v.at[pl.ds(k * _GCHUNK, _GCHUNK)]],
            rows_v.at[pl.ds(k * _GCHUNK, _GCHUNK)],
            sem,
        )
        for k in range(_NGCHUNK)
    ]
    for c in copies:
        c.wait()

    def body(b, carry):
        row0 = b * CTX
        for c in range(D // LANES):
            sl = pl.ds(c * LANES, LANES)
            acc = rows_v[row0, sl]
            for j in range(1, CTX):
                acc = acc + rows_v[row0 + j, sl]
            mean_v[b, sl] = acc * jnp.float32(1.0 / CTX)
        return carry

    lax.fori_loop(0, _B_PER_W, body, 0)
    pltpu.sync_copy(mean_v, out_hbm.at[pl.ds(wid * _B_PER_W, _B_PER_W)])


_gather_mean = functools.partial(
    pl.kernel,
    out_type=jax.ShapeDtypeStruct((BATCH, D), jnp.float32),
    mesh=plsc.VectorSubcoreMesh(core_axis_name="c", subcore_axis_name="s"),
    scratch_types=[
        pltpu.VMEM((_IDS_PER_W,), jnp.int32),
        pltpu.VMEM((_IDS_PER_W, D), jnp.float32),
        pltpu.VMEM((_B_PER_W, D), jnp.float32),
        pltpu.SemaphoreType.DMA,
    ],
    compiler_params=pltpu.CompilerParams(use_tc_tiling_on_sc=False),
)(_gather_mean_body)


_VB = 2048


def _proj_body(wt_ref, x_ref, out_ref):
    out_ref[...] = lax.dot_general(
        wt_ref[...].astype(jnp.bfloat16),
        x_ref[...],
        (((0,), (1,)), ((), ())),
        preferred_element_type=jnp.float32,
    )


def _project_t(wt, x):
    return pl.pallas_call(
        _proj_body,
        grid=(pl.cdiv(VOCAB, _VB),),
        in_specs=[
            pl.BlockSpec((D, _VB), lambda i: (0, i)),
            pl.BlockSpec((BATCH, D), lambda i: (0, 0)),
        ],
        out_specs=pl.BlockSpec((_VB, BATCH), lambda i: (i, 0)),
        out_shape=jax.ShapeDtypeStruct((VOCAB, BATCH), jnp.float32),
        compiler_params=pltpu.CompilerParams(
            dimension_semantics=("arbitrary",),
            vmem_limit_bytes=100 * 1024 * 1024,
        ),
    )(wt, x)


def kernel(context_ids, emb_table, out_weight):
    ids_flat = context_ids.reshape(BATCH * CTX).astype(jnp.int32)
    means = _gather_mean(ids_flat, emb_table)
    x = means.astype(jnp.bfloat16)
    logits_t = _project_t(out_weight.T, x)
    return logits_t.T


# speedup vs baseline: 2.8045x; 1.0102x over previous
import functools

import jax
import jax.numpy as jnp
from jax import lax
from jax.experimental import pallas as pl
from jax.experimental.pallas import tpu as pltpu
from jax.experimental.pallas import tpu_sc as plsc

VOCAB = 100000
D = 64
BATCH = 1024
CTX = 10
LANES = 16
NIDS = BATCH * CTX

_INFO = plsc.get_sparse_core_info()
_NC, _NS = _INFO.num_cores, _INFO.num_subcores
_NW = _NC * _NS
_D_PER_W = D // _NW
_GCHUNK = 128
_NGCHUNK = NIDS // _GCHUNK


def _gather_mean_body(ids_hbm, et_hbm, out_hbm, idx_v, buf_v, mean_v, sem):
    wid = lax.axis_index("s") * _NC + lax.axis_index("c")
    pltpu.sync_copy(ids_hbm, idx_v)

    def one_dim(t, carry):
        d = wid * _D_PER_W + t
        row = et_hbm.at[d]
        copies = [
            pltpu.async_copy(
                row.at[idx_v.at[pl.ds(k * _GCHUNK, _GCHUNK)]],
                buf_v.at[pl.ds(k * _GCHUNK, _GCHUNK)],
                sem,
            )
            for k in range(_NGCHUNK)
        ]
        for c in copies:
            c.wait()

        def chunk(c, carry2):
            sl = pl.ds(c * LANES, LANES)
            acc = buf_v[pl.ds(c * LANES, LANES)]
            for j in range(1, CTX):
                acc = acc + buf_v[pl.ds(j * BATCH + c * LANES, LANES)]
            mean_v[t, sl] = acc * jnp.float32(1.0 / CTX)
            return carry2

        lax.fori_loop(0, BATCH // LANES, chunk, 0)
        return carry

    lax.fori_loop(0, _D_PER_W, one_dim, 0, unroll=True)
    pltpu.sync_copy(mean_v, out_hbm.at[pl.ds(wid * _D_PER_W, _D_PER_W)])


_gather_mean_t = functools.partial(
    pl.kernel,
    out_type=jax.ShapeDtypeStruct((D, BATCH), jnp.float32),
    mesh=plsc.VectorSubcoreMesh(core_axis_name="c", subcore_axis_name="s"),
    scratch_types=[
        pltpu.VMEM((NIDS,), jnp.int32),
        pltpu.VMEM((NIDS,), jnp.float32),
        pltpu.VMEM((_D_PER_W, BATCH), jnp.float32),
        pltpu.SemaphoreType.DMA,
    ],
    compiler_params=pltpu.CompilerParams(use_tc_tiling_on_sc=False),
)(_gather_mean_body)


_VB = 2048


def _proj_body(wt_ref, x_ref, out_ref):
    out_ref[...] = lax.dot_general(
        wt_ref[...].astype(jnp.bfloat16),
        x_ref[...],
        (((0,), (0,)), ((), ())),
        preferred_element_type=jnp.float32,
    )


def _project_t(wt, xt):
    return pl.pallas_call(
        _proj_body,
        grid=(pl.cdiv(VOCAB, _VB),),
        in_specs=[
            pl.BlockSpec((D, _VB), lambda i: (0, i)),
            pl.BlockSpec((D, BATCH), lambda i: (0, 0)),
        ],
        out_specs=pl.BlockSpec((_VB, BATCH), lambda i: (i, 0)),
        out_shape=jax.ShapeDtypeStruct((VOCAB, BATCH), jnp.float32),
        compiler_params=pltpu.CompilerParams(
            dimension_semantics=("arbitrary",),
            vmem_limit_bytes=100 * 1024 * 1024,
        ),
    )(wt, xt)


def kernel(context_ids, emb_table, out_weight):
    ids_cm = context_ids.T.reshape(NIDS).astype(jnp.int32)
    means_t = _gather_mean_t(ids_cm, emb_table.T)
    xt = means_t.astype(jnp.bfloat16)
    logits_t = _project_t(out_weight.T, xt)
    return logits_t.T
